# baseline (speedup 1.0000x reference)
import functools

import jax
import jax.numpy as jnp
from jax import lax
from jax.experimental import pallas as pl
from jax.experimental.pallas import tpu as pltpu
from jax.experimental.pallas import tpu_sc as plsc

_B, _P, _K = 32, 512, 20
_L = 16


def _tc_front_kernel(shift_ref, pos_ref, W1a_ref, b1a_ref, W1b_ref, b1b_ref,
                     W2_ref, b2_ref, Wh_ref, bh_ref,
                     idx_ref, y_ref, z_ref):
    f32 = jnp.float32
    x = pos_ref[0] + shift_ref[0, 0]
    iota_q = jax.lax.broadcasted_iota(jnp.int32, (_P, _P), 1)

    def dot(a, b):
        return jax.lax.dot_general(a, b, (((1,), (0,)), ((), ())),
                                   preferred_element_type=f32)

    def pairwise_d2(feat):
        sq = jnp.sum(feat * feat, axis=1, keepdims=True)
        g = jax.lax.dot_general(feat, feat, (((1,), (1,)), ((), ())),
                                preferred_element_type=f32)
        return sq + sq.reshape(1, _P) - 2.0 * g

    def argmin_step(d2c):
        m = jnp.min(d2c, axis=1, keepdims=True)
        am = jnp.min(jnp.where(d2c == m, iota_q, _P), axis=1, keepdims=True)
        onehot_b = iota_q == am
        d2c = jnp.where(onehot_b, jnp.inf, d2c)
        return d2c, am, onehot_b.astype(f32)

    W1a_top = W1a_ref[0:3, :]
    W1a_bot = W1a_ref[3:6, :]
    c1 = dot(x, W1a_top - W1a_bot) + b1a_ref[0]

    d2c = pairwise_d2(x)
    f1 = jnp.full((_P, 64), -jnp.inf, dtype=f32)
    for _ in range(_K):
        d2c, _, onehot = argmin_step(d2c)
        xj = dot(onehot, x)
        f1 = jnp.maximum(f1, dot(jax.nn.relu(c1 + dot(xj, W1a_bot)),
                                 W1b_ref[...]))
    f1 = f1 + b1b_ref[0]

    d2c = pairwise_d2(f1)
    for t in range(_K):
        d2c, am, _ = argmin_step(d2c)
        idx_ref[0, :, t:t + 1] = am

    W2_top = W2_ref[0:64, :]
    W2_bot = W2_ref[64:128, :]
    c2 = dot(f1, W2_top - W2_bot) + b2_ref[0]
    y_ref[0] = dot(f1, W2_bot)
    z_ref[0] = (dot(f1, Wh_ref[0:64, :]) + dot(c2, Wh_ref[64:192, :])
                + bh_ref[0])


def _sc_gather_max(y_hbm, idx_hbm, out_hbm, y_v, idx_v, tbuf, sem):
    wid = lax.axis_index("s") * 2 + lax.axis_index("c")
    pltpu.sync_copy(y_hbm.at[wid], y_v)
    pltpu.sync_copy(idx_hbm.at[wid], idx_v)
    lanes = lax.iota(jnp.int32, _L)

    def point_chunk(pc, _):
        base = pc * _L
        pt20 = (base + lanes) * _K
        addr = [plsc.load_gather(idx_v, [pt20 + t]) * 128
                for t in range(_K)]

        def chan(c, _):
            cs = jnp.full((_L,), c, jnp.int32)
            acc = plsc.load_gather(y_v, [addr[0] + cs])
            for t in range(1, _K):
                acc = jnp.maximum(acc, plsc.load_gather(y_v, [addr[t] + cs]))
            tbuf[pl.ds(c * _L, _L)] = acc
            return 0

        lax.fori_loop(0, 128, chan, 0)
        copy = pltpu.make_async_copy(
            tbuf, out_hbm.at[wid].at[pl.ds(base * 128, 128 * _L)], sem)
        copy.start()
        copy.wait()
        return 0

    lax.fori_loop(0, _P // _L, point_chunk, 0)


def _tc_tail_kernel(z_ref, t_ref, Wh_ref, out_ref):
    s = jax.lax.dot_general(t_ref[0], Wh_ref[64:192, :],
                            (((0,), (0,)), ((), ())),
                            preferred_element_type=jnp.float32)
    out_ref[0] = jnp.max(z_ref[0] + s, axis=0, keepdims=True)


def kernel(pos, batch, W1a, b1a, W1b, b1b, W2, b2, Wh, bh):
    nb = _B
    pp = pos.shape[0] // nb
    shift = (batch[-1].astype(jnp.int32) + 1 - nb).astype(pos.dtype)
    posb = pos.reshape(nb, pp, 3)
    shift2d = shift.reshape(1, 1)

    full = lambda shape: pl.BlockSpec(shape, lambda g: (0,) * len(shape))
    idx2, y, z = pl.pallas_call(
        _tc_front_kernel,
        grid=(nb,),
        in_specs=[
            full((1, 1)),
            pl.BlockSpec((1, pp, 3), lambda g: (g, 0, 0)),
            full((6, 64)), full((1, 64)),
            full((64, 64)), full((1, 64)),
            full((128, 128)), full((1, 128)),
            full((192, 128)), full((1, 128)),
        ],
        out_specs=[
            pl.BlockSpec((1, pp, _K), lambda g: (g, 0, 0)),
            pl.BlockSpec((1, pp, 128), lambda g: (g, 0, 0)),
            pl.BlockSpec((1, pp, 128), lambda g: (g, 0, 0)),
        ],
        out_shape=[
            jax.ShapeDtypeStruct((nb, pp, _K), jnp.int32),
            jax.ShapeDtypeStruct((nb, pp, 128), jnp.float32),
            jax.ShapeDtypeStruct((nb, pp, 128), jnp.float32),
        ],
        compiler_params=pltpu.CompilerParams(
            dimension_semantics=("parallel",)),
    )(shift2d, posb, W1a, b1a.reshape(1, 64), W1b, b1b.reshape(1, 64),
      W2, b2.reshape(1, 128), Wh, bh.reshape(1, 128))

    mesh = plsc.VectorSubcoreMesh(core_axis_name="c", subcore_axis_name="s")
    t_flat = pl.kernel(
        _sc_gather_max,
        mesh=mesh,
        out_type=jax.ShapeDtypeStruct((nb, pp * 128), jnp.float32),
        scratch_types=[
            pltpu.VMEM((pp * 128,), jnp.float32),
            pltpu.VMEM((pp * _K,), jnp.int32),
            pltpu.VMEM((128 * _L,), jnp.float32),
            pltpu.SemaphoreType.DMA,
        ],
        compiler_params=pltpu.CompilerParams(needs_layout_passes=False),
    )(y.reshape(nb, pp * 128), idx2.reshape(nb, pp * _K))
    t_cm = t_flat.reshape(nb, pp // _L, 128, _L).transpose(0, 2, 1, 3)
    t_cm = t_cm.reshape(nb, 128, pp)

    out = pl.pallas_call(
        _tc_tail_kernel,
        grid=(nb,),
        in_specs=[
            pl.BlockSpec((1, pp, 128), lambda g: (g, 0, 0)),
            pl.BlockSpec((1, 128, pp), lambda g: (g, 0, 0)),
            full((192, 128)),
        ],
        out_specs=pl.BlockSpec((1, 1, 128), lambda g: (g, 0, 0)),
        out_shape=jax.ShapeDtypeStruct((nb, 1, 128), jnp.float32),
        compiler_params=pltpu.CompilerParams(
            dimension_semantics=("parallel",)),
    )(z, t_cm, Wh)
    return out.reshape(nb, 128)


# speedup vs baseline: 1.0230x; 1.0230x over previous
import functools

import jax
import jax.numpy as jnp
from jax import lax
from jax.experimental import pallas as pl
from jax.experimental.pallas import tpu as pltpu
from jax.experimental.pallas import tpu_sc as plsc

_B, _P, _K = 32, 512, 20
_L = 16


def _tc_front_kernel(shift_ref, pos_ref, W1a_ref, b1a_ref, W1b_ref, b1b_ref,
                     W2_ref, b2_ref, Wh_ref, bh_ref,
                     idx_ref, y_ref, z_ref):
    f32 = jnp.float32
    x = pos_ref[0] + shift_ref[0, 0]
    iota_q = jax.lax.broadcasted_iota(jnp.int32, (_P, _P), 1)

    def dot(a, b):
        return jax.lax.dot_general(a, b, (((1,), (0,)), ((), ())),
                                   preferred_element_type=f32)

    def pairwise_d2(feat):
        sq = jnp.sum(feat * feat, axis=1, keepdims=True)
        g = jax.lax.dot_general(feat, feat, (((1,), (1,)), ((), ())),
                                preferred_element_type=f32)
        return sq + sq.reshape(1, _P) - 2.0 * g

    def argmin_step(d2c):
        m = jnp.min(d2c, axis=1, keepdims=True)
        am = jnp.min(jnp.where(d2c == m, iota_q, _P), axis=1, keepdims=True)
        onehot_b = iota_q == am
        d2c = jnp.where(onehot_b, jnp.inf, d2c)
        return d2c, am, onehot_b.astype(f32)

    W1a_top = W1a_ref[0:3, :]
    W1a_bot = W1a_ref[3:6, :]
    c1 = dot(x, W1a_top - W1a_bot) + b1a_ref[0]

    d2c = pairwise_d2(x)
    f1 = jnp.full((_P, 64), -jnp.inf, dtype=f32)
    for _ in range(_K):
        d2c, _, onehot = argmin_step(d2c)
        xj = dot(onehot, x)
        f1 = jnp.maximum(f1, dot(jax.nn.relu(c1 + dot(xj, W1a_bot)),
                                 W1b_ref[...]))
    f1 = f1 + b1b_ref[0]

    d2c = pairwise_d2(f1)
    for t in range(_K):
        d2c, am, _ = argmin_step(d2c)
        idx_ref[0, :, t:t + 1] = am

    W2_top = W2_ref[0:64, :]
    W2_bot = W2_ref[64:128, :]
    c2 = dot(f1, W2_top - W2_bot) + b2_ref[0]
    y_ref[0] = dot(f1, W2_bot)
    z_ref[0] = (dot(f1, Wh_ref[0:64, :]) + dot(c2, Wh_ref[64:192, :])
                + bh_ref[0])


def _sc_gather_max(y_hbm, idx_hbm, out_hbm, y_v, idx_v, tbuf, sems):
    wid = lax.axis_index("s") * 2 + lax.axis_index("c")
    pltpu.sync_copy(y_hbm.at[wid], y_v)
    pltpu.sync_copy(idx_hbm.at[wid], idx_v)
    lanes = lax.iota(jnp.int32, _L)

    def point_chunk(pc, _):
        base = pc * _L
        pt20 = (base + lanes) * _K
        addr = [plsc.load_gather(idx_v, [pt20 + t]) * 128
                for t in range(_K)]
        buf = jax.lax.rem(pc, 2)

        @pl.when(pc >= 2)
        def _():
            pltpu.make_async_copy(
                tbuf.at[pl.ds(buf * 2048, 2048)],
                out_hbm.at[wid].at[pl.ds(base * 128, 128 * _L)],
                sems.at[buf]).wait()

        @plsc.parallel_loop(0, 128, 1, unroll=4)
        def chan(c):
            cs = jnp.full((_L,), c, jnp.int32)
            vals = [plsc.load_gather(y_v, [addr[t] + cs]) for t in range(_K)]
            while len(vals) > 1:
                nxt = [jnp.maximum(vals[i], vals[i + 1])
                       for i in range(0, len(vals) - 1, 2)]
                if len(vals) % 2:
                    nxt.append(vals[-1])
                vals = nxt
            tbuf[pl.ds(buf * 2048 + c * _L, _L)] = vals[0]

        pltpu.make_async_copy(
            tbuf.at[pl.ds(buf * 2048, 2048)],
            out_hbm.at[wid].at[pl.ds(base * 128, 128 * _L)],
            sems.at[buf]).start()
        return 0

    nchunk = _P // _L
    lax.fori_loop(0, nchunk, point_chunk, 0)
    for b in range(2):
        pltpu.make_async_copy(
            tbuf.at[pl.ds(b * 2048, 2048)],
            out_hbm.at[wid].at[pl.ds((nchunk - 2 + b) * 128 * _L, 128 * _L)],
            sems.at[b]).wait()


def _tc_tail_kernel(z_ref, t_ref, Wh_ref, out_ref):
    s = jax.lax.dot_general(t_ref[0], Wh_ref[64:192, :],
                            (((0,), (0,)), ((), ())),
                            preferred_element_type=jnp.float32)
    out_ref[0] = jnp.max(z_ref[0] + s, axis=0, keepdims=True)


def kernel(pos, batch, W1a, b1a, W1b, b1b, W2, b2, Wh, bh):
    nb = _B
    pp = pos.shape[0] // nb
    shift = (batch[-1].astype(jnp.int32) + 1 - nb).astype(pos.dtype)
    posb = pos.reshape(nb, pp, 3)
    shift2d = shift.reshape(1, 1)

    full = lambda shape: pl.BlockSpec(shape, lambda g: (0,) * len(shape))
    idx2, y, z = pl.pallas_call(
        _tc_front_kernel,
        grid=(nb,),
        in_specs=[
            full((1, 1)),
            pl.BlockSpec((1, pp, 3), lambda g: (g, 0, 0)),
            full((6, 64)), full((1, 64)),
            full((64, 64)), full((1, 64)),
            full((128, 128)), full((1, 128)),
            full((192, 128)), full((1, 128)),
        ],
        out_specs=[
            pl.BlockSpec((1, pp, _K), lambda g: (g, 0, 0)),
            pl.BlockSpec((1, pp, 128), lambda g: (g, 0, 0)),
            pl.BlockSpec((1, pp, 128), lambda g: (g, 0, 0)),
        ],
        out_shape=[
            jax.ShapeDtypeStruct((nb, pp, _K), jnp.int32),
            jax.ShapeDtypeStruct((nb, pp, 128), jnp.float32),
            jax.ShapeDtypeStruct((nb, pp, 128), jnp.float32),
        ],
        compiler_params=pltpu.CompilerParams(
            dimension_semantics=("parallel",)),
    )(shift2d, posb, W1a, b1a.reshape(1, 64), W1b, b1b.reshape(1, 64),
      W2, b2.reshape(1, 128), Wh, bh.reshape(1, 128))

    mesh = plsc.VectorSubcoreMesh(core_axis_name="c", subcore_axis_name="s")
    t_flat = pl.kernel(
        _sc_gather_max,
        mesh=mesh,
        out_type=jax.ShapeDtypeStruct((nb, pp * 128), jnp.float32),
        scratch_types=[
            pltpu.VMEM((pp * 128,), jnp.float32),
            pltpu.VMEM((pp * _K,), jnp.int32),
            pltpu.VMEM((2 * 128 * _L,), jnp.float32),
            pltpu.SemaphoreType.DMA((2,)),
        ],
        compiler_params=pltpu.CompilerParams(needs_layout_passes=False),
    )(y.reshape(nb, pp * 128), idx2.reshape(nb, pp * _K))
    t_cm = t_flat.reshape(nb, pp // _L, 128, _L).transpose(0, 2, 1, 3)
    t_cm = t_cm.reshape(nb, 128, pp)

    out = pl.pallas_call(
        _tc_tail_kernel,
        grid=(nb,),
        in_specs=[
            pl.BlockSpec((1, pp, 128), lambda g: (g, 0, 0)),
            pl.BlockSpec((1, 128, pp), lambda g: (g, 0, 0)),
            full((192, 128)),
        ],
        out_specs=pl.BlockSpec((1, 1, 128), lambda g: (g, 0, 0)),
        out_shape=jax.ShapeDtypeStruct((nb, 1, 128), jnp.float32),
        compiler_params=pltpu.CompilerParams(
            dimension_semantics=("parallel",)),
    )(z, t_cm, Wh)
    return out.reshape(nb, 128)


# speedup vs baseline: 2.2979x; 2.2462x over previous
import jax
import jax.numpy as jnp
from jax.experimental import pallas as pl
from jax.experimental.pallas import tpu as pltpu

_B, _P, _K = 32, 512, 20


def _graph_kernel(shift_ref, pos_ref, W1a_ref, b1a_ref, W1b_ref, b1b_ref,
                  W2_ref, b2_ref, Wh_ref, bh_ref, out_ref):
    f32 = jnp.float32
    iota_q = jax.lax.broadcasted_iota(jnp.int32, (_P, _P), 1).astype(f32)

    def dot(a, b):
        return jax.lax.dot_general(a, b, (((1,), (0,)), ((), ())),
                                   preferred_element_type=f32)

    def pairwise_d2(feat):
        sq = jnp.sum(feat * feat, axis=1, keepdims=True)
        g = jax.lax.dot_general(feat, feat, (((1,), (1,)), ((), ())),
                                preferred_element_type=f32)
        return sq + sq.reshape(1, _P) - 2.0 * g

    def knn_max(d2, table, msg_fn, out_dim):
        acc0 = jnp.full((_P, out_dim), -jnp.inf, dtype=f32)

        d2c, acc = d2, acc0
        for _ in range(_K):
            m = jnp.min(d2c, axis=1, keepdims=True)
            am = jnp.min(jnp.where(d2c == m, iota_q, float(_P)), axis=1,
                         keepdims=True)
            onehot_b = iota_q == am
            onehot = onehot_b.astype(f32)
            gathered = dot(onehot, table)
            acc = jnp.maximum(acc, msg_fn(gathered))
            d2c = jnp.where(onehot_b, jnp.inf, d2c)
        return acc

    for i in range(pos_ref.shape[0]):
        x = pos_ref[i] + shift_ref[0, 0]

        W1a_top = W1a_ref[0:3, :]
        W1a_bot = W1a_ref[3:6, :]
        c1 = dot(x, W1a_top - W1a_bot) + b1a_ref[0]

        def msg1(xj, c1=c1):
            return dot(jax.nn.relu(c1 + dot(xj, W1a_bot)), W1b_ref[...])

        f1 = knn_max(pairwise_d2(x), x, msg1, 64) + b1b_ref[0]

        W2_top = W2_ref[0:64, :]
        W2_bot = W2_ref[64:128, :]
        c2 = dot(f1, W2_top - W2_bot) + b2_ref[0]

        def msg2(fj):
            return dot(fj, W2_bot)

        f2 = c2 + knn_max(pairwise_d2(f1), f1, msg2, 128)

        h = dot(f1, Wh_ref[0:64, :]) + dot(f2, Wh_ref[64:192, :]) + bh_ref[0]
        out_ref[i] = jnp.max(h, axis=0, keepdims=True)


def kernel(pos, batch, W1a, b1a, W1b, b1b, W2, b2, Wh, bh):
    nb = _B
    pp = pos.shape[0] // nb
    shift = (batch[-1].astype(jnp.int32) + 1 - nb).astype(pos.dtype)
    posb = pos.reshape(nb, pp, 3)
    shift2d = shift.reshape(1, 1)

    full = lambda shape: pl.BlockSpec(shape, lambda g: (0,) * len(shape))
    gpb = 2
    out = pl.pallas_call(
        _graph_kernel,
        grid=(nb // gpb,),
        in_specs=[
            full((1, 1)),
            pl.BlockSpec((gpb, pp, 3), lambda g: (g, 0, 0)),
            full((6, 64)), full((1, 64)),
            full((64, 64)), full((1, 64)),
            full((128, 128)), full((1, 128)),
            full((192, 128)), full((1, 128)),
        ],
        out_specs=pl.BlockSpec((gpb, 1, 128), lambda g: (g, 0, 0)),
        out_shape=jax.ShapeDtypeStruct((nb, 1, 128), jnp.float32),
        compiler_params=pltpu.CompilerParams(
            dimension_semantics=("parallel",)),
    )(shift2d, posb, W1a, b1a.reshape(1, 64), W1b, b1b.reshape(1, 64),
      W2, b2.reshape(1, 128), Wh, bh.reshape(1, 128))
    return out.reshape(nb, 128)


# speedup vs baseline: 2.3247x; 1.0117x over previous
import jax
import jax.numpy as jnp
from jax.experimental import pallas as pl
from jax.experimental.pallas import tpu as pltpu

_B, _P, _K = 32, 512, 20


def _graph_kernel(shift_ref, pos_ref, W1a_ref, b1a_ref, W1b_ref, b1b_ref,
                  W2_ref, b2_ref, Wh_ref, bh_ref, out_ref):
    f32 = jnp.float32
    iota_q = jax.lax.broadcasted_iota(jnp.int32, (_P, _P), 1).astype(f32)

    def dot(a, b):
        return jax.lax.dot_general(a, b, (((1,), (0,)), ((), ())),
                                   preferred_element_type=f32)

    def pairwise_d2(feat):
        sq = jnp.sum(feat * feat, axis=1, keepdims=True)
        g = jax.lax.dot_general(feat, feat, (((1,), (1,)), ((), ())),
                                preferred_element_type=f32)
        return sq + sq.reshape(1, _P) - 2.0 * g

    def knn_max_multi(d2s, tables, msg_fns, out_dim):
        n = len(d2s)
        d2cs = list(d2s)
        accs = [jnp.full((_P, out_dim), -jnp.inf, dtype=f32)] * n
        for _ in range(_K):
            for i in range(n):
                d2c = d2cs[i]
                m = jnp.min(d2c, axis=1, keepdims=True)
                am = jnp.min(jnp.where(d2c == m, iota_q, float(_P)), axis=1,
                             keepdims=True)
                onehot_b = iota_q == am
                onehot = onehot_b.astype(f32)
                gathered = dot(onehot, tables[i])
                accs[i] = jnp.maximum(accs[i], msg_fns[i](gathered))
                d2cs[i] = jnp.where(onehot_b, jnp.inf, d2c)
        return accs

    ngr = pos_ref.shape[0]
    xs = [pos_ref[i] + shift_ref[0, 0] for i in range(ngr)]

    W1a_top = W1a_ref[0:3, :]
    W1a_bot = W1a_ref[3:6, :]
    c1s = [dot(x, W1a_top - W1a_bot) + b1a_ref[0] for x in xs]

    def mk_msg1(c1):
        return lambda xj: dot(jax.nn.relu(c1 + dot(xj, W1a_bot)),
                              W1b_ref[...])

    f1s = knn_max_multi([pairwise_d2(x) for x in xs], xs,
                        [mk_msg1(c1) for c1 in c1s], 64)
    f1s = [f1 + b1b_ref[0] for f1 in f1s]

    W2_top = W2_ref[0:64, :]
    W2_bot = W2_ref[64:128, :]
    c2s = [dot(f1, W2_top - W2_bot) + b2_ref[0] for f1 in f1s]
    msg2 = lambda fj: dot(fj, W2_bot)

    f2s = knn_max_multi([pairwise_d2(f1) for f1 in f1s], f1s,
                        [msg2] * ngr, 128)

    for i in range(ngr):
        h = (dot(f1s[i], Wh_ref[0:64, :])
             + dot(c2s[i] + f2s[i], Wh_ref[64:192, :]) + bh_ref[0])
        out_ref[i] = jnp.max(h, axis=0, keepdims=True)


def kernel(pos, batch, W1a, b1a, W1b, b1b, W2, b2, Wh, bh):
    nb = _B
    pp = pos.shape[0] // nb
    shift = (batch[-1].astype(jnp.int32) + 1 - nb).astype(pos.dtype)
    posb = pos.reshape(nb, pp, 3)
    shift2d = shift.reshape(1, 1)

    full = lambda shape: pl.BlockSpec(shape, lambda g: (0,) * len(shape))
    gpb = 2
    out = pl.pallas_call(
        _graph_kernel,
        grid=(nb // gpb,),
        in_specs=[
            full((1, 1)),
            pl.BlockSpec((gpb, pp, 3), lambda g: (g, 0, 0)),
            full((6, 64)), full((1, 64)),
            full((64, 64)), full((1, 64)),
            full((128, 128)), full((1, 128)),
            full((192, 128)), full((1, 128)),
        ],
        out_specs=pl.BlockSpec((gpb, 1, 128), lambda g: (g, 0, 0)),
        out_shape=jax.ShapeDtypeStruct((nb, 1, 128), jnp.float32),
        compiler_params=pltpu.CompilerParams(
            dimension_semantics=("parallel",)),
    )(shift2d, posb, W1a, b1a.reshape(1, 64), W1b, b1b.reshape(1, 64),
      W2, b2.reshape(1, 128), Wh, bh.reshape(1, 128))
    return out.reshape(nb, 128)
